# baseline (speedup 1.0000x reference)
import functools

import jax
import jax.numpy as jnp
from jax import lax
from jax.experimental import pallas as pl
from jax.experimental.pallas import tpu as pltpu
from jax.experimental.pallas import tpu_sc as plsc

N = 10000
E = 320000
D = 128

NC = 2
NS = 16
CK = 64
CH = 160
NB = 4
NH = 4
CH2 = CH // NH
EPW = CH * CK
E_PAD = NC * NS * EPW
NP = 10112
RPT = NP // NS


def _sc_agg(h, src_p, dst_p, zinit):
    mesh = plsc.VectorSubcoreMesh(core_axis_name="c", subcore_axis_name="s")

    @functools.partial(
        pl.kernel,
        mesh=mesh,
        out_type=jax.ShapeDtypeStruct((NC, NP, D), jnp.float32),
        scratch_types=[
            pltpu.VMEM((CH2, CK), jnp.int32),
            pltpu.VMEM((CH2, CK), jnp.int32),
            pltpu.VMEM((NB, CK, D), jnp.float32),
            pltpu.VMEM_SHARED((NP, D), jnp.float32),
            pltpu.SemaphoreType.DMA,
        ],
    )
    def k(h_hbm, src_hbm, dst_hbm, z_hbm, out_hbm, src_v, dst_v, rows_v, agg_sh,
          sem0):
        c = lax.axis_index("c")
        s = lax.axis_index("s")
        pltpu.sync_copy(z_hbm.at[pl.ds(s * RPT, RPT)], agg_sh.at[pl.ds(s * RPT, RPT)])
        plsc.subcore_barrier()

        for ph in range(NH):
            pltpu.sync_copy(src_hbm.at[c, s, ph], src_v)
            pltpu.sync_copy(dst_hbm.at[c, s, ph], dst_v)

            for p in range(NB - 1):
                pltpu.async_copy(h_hbm.at[src_v.at[p]], rows_v.at[p], sem0)

            @pl.loop(0, CH2)
            def _(j):
                @pl.when(j + NB - 1 < CH2)
                def _():
                    pltpu.async_copy(h_hbm.at[src_v.at[j + NB - 1]],
                                     rows_v.at[lax.rem(j + NB - 1, NB)], sem0)
                pb = lax.rem(j, NB)
                pltpu.make_async_copy(h_hbm.at[pl.ds(0, CK)], rows_v.at[pb],
                                      sem0).wait()
                pltpu.sync_copy(rows_v.at[pb], agg_sh.at[dst_v.at[j]],
                                add=True)

        plsc.subcore_barrier()
        pltpu.sync_copy(agg_sh.at[pl.ds(s * RPT, RPT)],
                        out_hbm.at[c, pl.ds(s * RPT, RPT)])

    return k(h, src_p, dst_p, zinit)


def _tc_mlp_bn(h, a0, a1, W1, b1, W2, b2, g, bb, relu_out):
    def body(h_ref, a0_ref, a1_ref, w1_ref, b1_ref, w2_ref, b2_ref, g_ref,
             bb_ref, o_ref):
        z = h_ref[...] + a0_ref[...] + a1_ref[...]
        t = jnp.dot(z, w1_ref[...], preferred_element_type=jnp.float32) + b1_ref[...]
        t = jnp.maximum(t, 0.0)
        u = jnp.dot(t, w2_ref[...], preferred_element_type=jnp.float32) + b2_ref[...]
        mu = jnp.mean(u, axis=0, keepdims=True)
        var = jnp.mean(jnp.square(u - mu), axis=0, keepdims=True)
        o = g_ref[...] * (u - mu) * lax.rsqrt(var + 1e-5) + bb_ref[...]
        if relu_out:
            o = jnp.maximum(o, 0.0)
        o_ref[...] = o

    return pl.pallas_call(
        body,
        out_shape=jax.ShapeDtypeStruct((N, D), jnp.float32),
    )(h, a0, a1, W1, b1, W2, b2, g, bb)


def kernel(x, edge_index, edge_attr, batch,
           W1_0, b1_0, W2_0, b2_0, bn_g_0, bn_b_0,
           W1_1, b1_1, W2_1, b2_1, bn_g_1, bn_b_1):
    x = x.astype(jnp.float32)
    pad = E_PAD - E
    perm = jnp.argsort(edge_index[0])
    src = jnp.concatenate([jnp.take(edge_index[0], perm), jnp.zeros((pad,), jnp.int32)])
    dst = jnp.concatenate([jnp.take(edge_index[1], perm), jnp.full((pad,), N, jnp.int32)])
    src_p = src.reshape(NC, NS, NH, CH2, CK)
    dst_p = dst.reshape(NC, NS, NH, CH2, CK)
    zinit = jnp.zeros((NP, D), jnp.float32)

    params = [
        (W1_0, b1_0, W2_0, b2_0, bn_g_0, bn_b_0),
        (W1_1, b1_1, W2_1, b2_1, bn_g_1, bn_b_1),
    ]
    h = x
    for layer, (W1, b1, W2, b2, g, bb) in enumerate(params):
        parts = _sc_agg(h, src_p, jnp.argsort(dst_p,axis=-1) if False else dst_p, zinit)
        h = _tc_mlp_bn(h, parts[0, :N], parts[1, :N], W1,
                       b1.reshape(1, D), W2, b2.reshape(1, D),
                       g.reshape(1, D), bb.reshape(1, D),
                       relu_out=(layer == 0))
    return h


# speedup vs baseline: 1.4090x; 1.4090x over previous
import functools

import jax
import jax.numpy as jnp
from jax import lax
from jax.experimental import pallas as pl
from jax.experimental.pallas import tpu as pltpu
from jax.experimental.pallas import tpu_sc as plsc

N = 10000
E = 320000
D = 128

NC = 2
NS = 16
CK = 64
CH = 160
NB = 4
NH = 4
CH2 = CH // NH
EPW = CH * CK
E_PAD = NC * NS * EPW
NP = 10112
RPT = NP // NS


def _sc_agg(h, src_p, dst_p, zinit):
    mesh = plsc.VectorSubcoreMesh(core_axis_name="c", subcore_axis_name="s")

    @functools.partial(
        pl.kernel,
        mesh=mesh,
        out_type=jax.ShapeDtypeStruct((NC, NP, D), jnp.float32),
        scratch_types=[
            pltpu.VMEM((CH2, CK), jnp.int32),
            pltpu.VMEM((CH2, CK), jnp.int32),
            pltpu.VMEM((NB, CK, D), jnp.float32),
            pltpu.VMEM_SHARED((NP, D), jnp.float32),
            pltpu.SemaphoreType.DMA,
        ],
    )
    def k(h_hbm, src_hbm, dst_hbm, z_hbm, out_hbm, src_v, dst_v, rows_v, agg_sh,
          sem0):
        c = lax.axis_index("c")
        s = lax.axis_index("s")
        pltpu.sync_copy(z_hbm.at[pl.ds(s * RPT, RPT)], agg_sh.at[pl.ds(s * RPT, RPT)])
        plsc.subcore_barrier()

        for ph in range(NH):
            pltpu.sync_copy(src_hbm.at[c, s, ph], src_v)
            pltpu.sync_copy(dst_hbm.at[c, s, ph], dst_v)

            for p in range(NB - 1):
                pltpu.async_copy(h_hbm.at[src_v.at[p]], rows_v.at[p], sem0)

            @pl.loop(0, CH2)
            def _(j):
                @pl.when(j + NB - 1 < CH2)
                def _():
                    pltpu.async_copy(h_hbm.at[src_v.at[j + NB - 1]],
                                     rows_v.at[lax.rem(j + NB - 1, NB)], sem0)
                pb = lax.rem(j, NB)
                pltpu.make_async_copy(h_hbm.at[pl.ds(0, CK)], rows_v.at[pb],
                                      sem0).wait()
                pltpu.sync_copy(rows_v.at[pb], agg_sh.at[dst_v.at[j]],
                                add=True)

        plsc.subcore_barrier()
        pltpu.sync_copy(agg_sh.at[pl.ds(s * RPT, RPT)],
                        out_hbm.at[c, pl.ds(s * RPT, RPT)])

    return k(h, src_p, dst_p, zinit)


def _tc_mlp_bn(h, a0, a1, W1, b1, W2, b2, g, bb, relu_out):
    def body(h_ref, a0_ref, a1_ref, w1_ref, b1_ref, w2_ref, b2_ref, g_ref,
             bb_ref, o_ref):
        z = h_ref[...] + a0_ref[...] + a1_ref[...]
        t = jnp.dot(z, w1_ref[...], preferred_element_type=jnp.float32) + b1_ref[...]
        t = jnp.maximum(t, 0.0)
        u = jnp.dot(t, w2_ref[...], preferred_element_type=jnp.float32) + b2_ref[...]
        mu = jnp.mean(u, axis=0, keepdims=True)
        var = jnp.mean(jnp.square(u - mu), axis=0, keepdims=True)
        o = g_ref[...] * (u - mu) * lax.rsqrt(var + 1e-5) + bb_ref[...]
        if relu_out:
            o = jnp.maximum(o, 0.0)
        o_ref[...] = o

    return pl.pallas_call(
        body,
        out_shape=jax.ShapeDtypeStruct((N, D), jnp.float32),
    )(h, a0, a1, W1, b1, W2, b2, g, bb)


def kernel(x, edge_index, edge_attr, batch,
           W1_0, b1_0, W2_0, b2_0, bn_g_0, bn_b_0,
           W1_1, b1_1, W2_1, b2_1, bn_g_1, bn_b_1):
    x = x.astype(jnp.float32)
    pad = E_PAD - E
    src = jnp.concatenate([edge_index[0], jnp.zeros((pad,), jnp.int32)])
    dst = jnp.concatenate([edge_index[1], jnp.full((pad,), N, jnp.int32)])
    src_p = src.reshape(NC, NS, NH, CH2, CK)
    dst_p = dst.reshape(NC, NS, NH, CH2, CK)
    zinit = jnp.zeros((NP, D), jnp.float32)

    params = [
        (W1_0, b1_0, W2_0, b2_0, bn_g_0, bn_b_0),
        (W1_1, b1_1, W2_1, b2_1, bn_g_1, bn_b_1),
    ]
    h = x
    for layer, (W1, b1, W2, b2, g, bb) in enumerate(params):
        parts = _sc_agg(h, src_p, jnp.argsort(dst_p,axis=-1) if False else dst_p, zinit)
        h = _tc_mlp_bn(h, parts[0, :N], parts[1, :N], W1,
                       b1.reshape(1, D), W2, b2.reshape(1, D),
                       g.reshape(1, D), bb.reshape(1, D),
                       relu_out=(layer == 0))
    return h


# speedup vs baseline: 2.0426x; 1.4497x over previous
import functools

import jax
import jax.numpy as jnp
from jax import lax
from jax.experimental import pallas as pl
from jax.experimental.pallas import tpu as pltpu
from jax.experimental.pallas import tpu_sc as plsc

N = 10000
E = 320000
D = 128

NC = 2
NS = 16
HALF = 5056
ACC = 5064
RPT = 320
PH = 448
NH = 46
EPT = PH * NH
E_PAD = NS * EPT
CKG = 32
CKS = 16
NCH = PH // CKG
TROWS = 632


def _sc_agg(h, src_p, dst_p, zinit):
    mesh = plsc.VectorSubcoreMesh(core_axis_name="c", subcore_axis_name="s")

    @functools.partial(
        pl.kernel,
        mesh=mesh,
        out_type=jax.ShapeDtypeStruct((NC, ACC, D), jnp.float32),
        scratch_types=[
            pltpu.VMEM((2, PH), jnp.int32),
            pltpu.VMEM((2, PH), jnp.int32),
            pltpu.VMEM((2, CKG, D), jnp.float32),
            pltpu.VMEM_SHARED((N, D), jnp.float32),
            pltpu.VMEM_SHARED((ACC, D), jnp.float32),
            pltpu.SemaphoreType.DMA,
            pltpu.SemaphoreType.DMA,
            pltpu.SemaphoreType.DMA,
        ],
    )
    def k(h_hbm, src_hbm, dst_hbm, z_hbm, out_hbm, src_v, dst_v, rows_v,
          tab_sh, acc_sh, isem, gsem, ssem):
        c = lax.axis_index("c")
        s = lax.axis_index("s")

        @pl.when(s < NS - 1)
        def _():
            pltpu.sync_copy(h_hbm.at[pl.ds(s * TROWS, TROWS)],
                            tab_sh.at[pl.ds(s * TROWS, TROWS)])

        @pl.when(s == NS - 1)
        def _():
            r = (NS - 1) * TROWS
            pltpu.sync_copy(h_hbm.at[pl.ds(r, N - r)], tab_sh.at[pl.ds(r, N - r)])

        @pl.when(s < NS - 1)
        def _():
            pltpu.sync_copy(z_hbm.at[pl.ds(s * RPT, RPT)],
                            acc_sh.at[pl.ds(s * RPT, RPT)])

        @pl.when(s == NS - 1)
        def _():
            rr = (NS - 1) * RPT
            pltpu.sync_copy(z_hbm.at[pl.ds(rr, ACC - rr)],
                            acc_sh.at[pl.ds(rr, ACC - rr)])
        pltpu.sync_copy(src_hbm.at[s, 0], src_v.at[0])
        pltpu.sync_copy(dst_hbm.at[c, s, 0], dst_v.at[0])
        plsc.subcore_barrier()

        def wait_idx(pb):
            pltpu.make_async_copy(src_hbm.at[0, 0], src_v.at[pb], isem).wait()
            pltpu.make_async_copy(src_hbm.at[0, 0], dst_v.at[pb], isem).wait()

        def wait_gather(b):
            pltpu.make_async_copy(h_hbm.at[pl.ds(0, CKG)], rows_v.at[b],
                                  gsem).wait()

        def drain_scatters(n):
            for _ in range(n):
                pltpu.make_async_copy(h_hbm.at[pl.ds(0, CKS)],
                                      rows_v.at[0, pl.ds(0, CKS)], ssem).wait()

        @pl.loop(0, NH)
        def _(ph):
            for pb in range(2):
                @pl.when(lax.rem(ph, 2) == pb)
                def _():
                    @pl.when(ph + 1 < NH)
                    def _():
                        pltpu.async_copy(src_hbm.at[s, ph + 1],
                                         src_v.at[1 - pb], isem)
                        pltpu.async_copy(dst_hbm.at[c, s, ph + 1],
                                         dst_v.at[1 - pb], isem)

                    pltpu.async_copy(
                        tab_sh.at[src_v.at[pb, pl.ds(0, CKG)]],
                        rows_v.at[0], gsem)

                    for kk in range(NCH):
                        b = kk % 2
                        if kk >= 1:
                            drain_scatters(2)
                        if kk + 1 < NCH:
                            pltpu.async_copy(
                                tab_sh.at[src_v.at[pb, pl.ds((kk + 1) * CKG, CKG)]],
                                rows_v.at[1 - b], gsem)
                        wait_gather(b)
                        for hh in range(2):
                            dvec = dst_v[pb, pl.ds(kk * CKG + hh * CKS, CKS)]
                            pltpu.async_copy(
                                rows_v.at[b, pl.ds(hh * CKS, CKS)],
                                acc_sh.at[dvec], ssem, add=True)
                    drain_scatters(2)

                    @pl.when(ph + 1 < NH)
                    def _():
                        wait_idx(1 - pb)

        plsc.subcore_barrier()

        @pl.when(s < NS - 1)
        def _():
            pltpu.sync_copy(acc_sh.at[pl.ds(s * RPT, RPT)],
                            out_hbm.at[c, pl.ds(s * RPT, RPT)])

        @pl.when(s == NS - 1)
        def _():
            rr = (NS - 1) * RPT
            pltpu.sync_copy(acc_sh.at[pl.ds(rr, ACC - rr)],
                            out_hbm.at[c, pl.ds(rr, ACC - rr)])

    return k(h, src_p, dst_p, zinit)


def _tc_mlp_bn(h, agg, W1, b1, W2, b2, g, bb, relu_out):
    def body(h_ref, a_ref, w1_ref, b1_ref, w2_ref, b2_ref, g_ref, bb_ref,
             o_ref):
        z = h_ref[...] + a_ref[...]
        t = jnp.dot(z, w1_ref[...], preferred_element_type=jnp.float32) + b1_ref[...]
        t = jnp.maximum(t, 0.0)
        u = jnp.dot(t, w2_ref[...], preferred_element_type=jnp.float32) + b2_ref[...]
        mu = jnp.mean(u, axis=0, keepdims=True)
        var = jnp.mean(jnp.square(u - mu), axis=0, keepdims=True)
        o = g_ref[...] * (u - mu) * lax.rsqrt(var + 1e-5) + bb_ref[...]
        if relu_out:
            o = jnp.maximum(o, 0.0)
        o_ref[...] = o

    return pl.pallas_call(
        body,
        out_shape=jax.ShapeDtypeStruct((N, D), jnp.float32),
    )(h, agg, W1, b1, W2, b2, g, bb)


def kernel(x, edge_index, edge_attr, batch,
           W1_0, b1_0, W2_0, b2_0, bn_g_0, bn_b_0,
           W1_1, b1_1, W2_1, b2_1, bn_g_1, bn_b_1):
    x = x.astype(jnp.float32)
    pad = E_PAD - E
    src = jnp.concatenate([edge_index[0], jnp.zeros((pad,), jnp.int32)])
    dst = jnp.concatenate([edge_index[1], jnp.full((pad,), N, jnp.int32)])
    dst0 = jnp.where(dst < HALF, dst, HALF)
    dst1 = jnp.where(dst >= HALF, dst - HALF, HALF)
    src_p = src.reshape(NS, NH, PH)
    dst_p = jnp.stack([dst0, dst1]).reshape(NC, NS, NH, PH)
    zinit = jnp.zeros((ACC, D), jnp.float32)

    params = [
        (W1_0, b1_0, W2_0, b2_0, bn_g_0, bn_b_0),
        (W1_1, b1_1, W2_1, b2_1, bn_g_1, bn_b_1),
    ]
    h = x
    for layer, (W1, b1, W2, b2, g, bb) in enumerate(params):
        parts = _sc_agg(h, src_p, dst_p, zinit)
        agg = jnp.concatenate([parts[0, :HALF], parts[1, :N - HALF]], axis=0)
        h = _tc_mlp_bn(h, agg, W1,
                       b1.reshape(1, D), W2, b2.reshape(1, D),
                       g.reshape(1, D), bb.reshape(1, D),
                       relu_out=(layer == 0))
    return h


# speedup vs baseline: 2.2201x; 1.0869x over previous
import functools

import jax
import jax.numpy as jnp
from jax import lax
from jax.experimental import pallas as pl
from jax.experimental.pallas import tpu as pltpu
from jax.experimental.pallas import tpu_sc as plsc

N = 10000
E = 320000
D = 128

NC = 2
NS = 16
HALF = 5056
ACC = 5080
RPT = 320
PH = 448
NH = 46
EPT = PH * NH
E_PAD = NS * EPT
CKG = 32
CKS = 16
NCH = PH // CKG
TROWS = 632


def _sc_agg(h, src_p, dst_p, zinit):
    mesh = plsc.VectorSubcoreMesh(core_axis_name="c", subcore_axis_name="s")

    @functools.partial(
        pl.kernel,
        mesh=mesh,
        out_type=jax.ShapeDtypeStruct((NC, ACC, D), jnp.float32),
        scratch_types=[
            pltpu.VMEM((2, PH), jnp.int32),
            pltpu.VMEM((2, PH), jnp.int32),
            pltpu.VMEM((2, CKG, D), jnp.float32),
            pltpu.VMEM_SHARED((N, D), jnp.float32),
            pltpu.VMEM_SHARED((ACC, D), jnp.float32),
            pltpu.SemaphoreType.DMA,
            pltpu.SemaphoreType.DMA,
            pltpu.SemaphoreType.DMA,
        ],
    )
    def k(h_hbm, src_hbm, dst_hbm, z_hbm, out_hbm, src_v, dst_v, rows_v,
          tab_sh, acc_sh, isem, gsem, ssem):
        c = lax.axis_index("c")
        s = lax.axis_index("s")

        @pl.when(s < NS - 1)
        def _():
            pltpu.sync_copy(h_hbm.at[pl.ds(s * TROWS, TROWS)],
                            tab_sh.at[pl.ds(s * TROWS, TROWS)])

        @pl.when(s == NS - 1)
        def _():
            r = (NS - 1) * TROWS
            pltpu.sync_copy(h_hbm.at[pl.ds(r, N - r)], tab_sh.at[pl.ds(r, N - r)])

        @pl.when(s < NS - 1)
        def _():
            pltpu.sync_copy(z_hbm.at[pl.ds(s * RPT, RPT)],
                            acc_sh.at[pl.ds(s * RPT, RPT)])

        @pl.when(s == NS - 1)
        def _():
            rr = (NS - 1) * RPT
            pltpu.sync_copy(z_hbm.at[pl.ds(rr, ACC - rr)],
                            acc_sh.at[pl.ds(rr, ACC - rr)])
        pltpu.sync_copy(src_hbm.at[s, 0], src_v.at[0])
        pltpu.sync_copy(dst_hbm.at[c, s, 0], dst_v.at[0])
        plsc.subcore_barrier()

        def wait_idx(pb):
            pltpu.make_async_copy(src_hbm.at[0, 0], src_v.at[pb], isem).wait()
            pltpu.make_async_copy(src_hbm.at[0, 0], dst_v.at[pb], isem).wait()

        def wait_gather(b):
            pltpu.make_async_copy(h_hbm.at[pl.ds(0, CKG)], rows_v.at[b],
                                  gsem).wait()

        def drain_scatters():
            pltpu.make_async_copy(h_hbm.at[pl.ds(0, CKG)], rows_v.at[0],
                                  ssem).wait()

        @pl.loop(0, NH)
        def _(ph):
            for pb in range(2):
                @pl.when(lax.rem(ph, 2) == pb)
                def _():
                    @pl.when(ph + 1 < NH)
                    def _():
                        pltpu.async_copy(src_hbm.at[s, ph + 1],
                                         src_v.at[1 - pb], isem)
                        pltpu.async_copy(dst_hbm.at[c, s, ph + 1],
                                         dst_v.at[1 - pb], isem)

                    pltpu.async_copy(
                        tab_sh.at[src_v.at[pb, pl.ds(0, CKG)]],
                        rows_v.at[0], gsem)

                    for kk in range(NCH):
                        b = kk % 2
                        if kk >= 1:
                            drain_scatters()
                        if kk + 1 < NCH:
                            pltpu.async_copy(
                                tab_sh.at[src_v.at[pb, pl.ds((kk + 1) * CKG, CKG)]],
                                rows_v.at[1 - b], gsem)
                        wait_gather(b)
                        for hh in range(2):
                            dvec = dst_v[pb, pl.ds(kk * CKG + hh * CKS, CKS)]
                            dvec = jnp.where(dvec == HALF, dvec + s, dvec)
                            pltpu.async_copy(
                                rows_v.at[b, pl.ds(hh * CKS, CKS)],
                                acc_sh.at[dvec], ssem, add=True)
                    drain_scatters()

                    @pl.when(ph + 1 < NH)
                    def _():
                        wait_idx(1 - pb)

        plsc.subcore_barrier()

        @pl.when(s < NS - 1)
        def _():
            pltpu.sync_copy(acc_sh.at[pl.ds(s * RPT, RPT)],
                            out_hbm.at[c, pl.ds(s * RPT, RPT)])

        @pl.when(s == NS - 1)
        def _():
            rr = (NS - 1) * RPT
            pltpu.sync_copy(acc_sh.at[pl.ds(rr, ACC - rr)],
                            out_hbm.at[c, pl.ds(rr, ACC - rr)])

    return k(h, src_p, dst_p, zinit)


def _tc_mlp_bn(h, agg, W1, b1, W2, b2, g, bb, relu_out):
    def body(h_ref, a_ref, w1_ref, b1_ref, w2_ref, b2_ref, g_ref, bb_ref,
             o_ref):
        z = h_ref[...] + a_ref[...]
        t = jnp.dot(z, w1_ref[...], preferred_element_type=jnp.float32) + b1_ref[...]
        t = jnp.maximum(t, 0.0)
        u = jnp.dot(t, w2_ref[...], preferred_element_type=jnp.float32) + b2_ref[...]
        mu = jnp.mean(u, axis=0, keepdims=True)
        var = jnp.mean(jnp.square(u - mu), axis=0, keepdims=True)
        o = g_ref[...] * (u - mu) * lax.rsqrt(var + 1e-5) + bb_ref[...]
        if relu_out:
            o = jnp.maximum(o, 0.0)
        o_ref[...] = o

    return pl.pallas_call(
        body,
        out_shape=jax.ShapeDtypeStruct((N, D), jnp.float32),
    )(h, agg, W1, b1, W2, b2, g, bb)


def kernel(x, edge_index, edge_attr, batch,
           W1_0, b1_0, W2_0, b2_0, bn_g_0, bn_b_0,
           W1_1, b1_1, W2_1, b2_1, bn_g_1, bn_b_1):
    x = x.astype(jnp.float32)
    pad = E_PAD - E
    src = jnp.concatenate([edge_index[0], jnp.zeros((pad,), jnp.int32)])
    dst = jnp.concatenate([edge_index[1], jnp.full((pad,), N, jnp.int32)])
    dst0 = jnp.where(dst < HALF, dst, HALF)
    dst1 = jnp.where(dst >= HALF, dst - HALF, HALF)
    src_p = src.reshape(NS, NH, PH)
    dst_p = jnp.stack([dst0, dst1]).reshape(NC, NS, NH, PH)
    zinit = jnp.zeros((ACC, D), jnp.float32)

    params = [
        (W1_0, b1_0, W2_0, b2_0, bn_g_0, bn_b_0),
        (W1_1, b1_1, W2_1, b2_1, bn_g_1, bn_b_1),
    ]
    h = x
    for layer, (W1, b1, W2, b2, g, bb) in enumerate(params):
        parts = _sc_agg(h, src_p, dst_p, zinit)
        agg = jnp.concatenate([parts[0, :HALF], parts[1, :N - HALF]], axis=0)
        h = _tc_mlp_bn(h, agg, W1,
                       b1.reshape(1, D), W2, b2.reshape(1, D),
                       g.reshape(1, D), bb.reshape(1, D),
                       relu_out=(layer == 0))
    return h
